# baseline (speedup 1.0000x reference)
import jax
import jax.numpy as jnp
from jax import lax
from jax.experimental import pallas as pl
from jax.experimental.pallas import tpu as pltpu
from jax.experimental.pallas import tpu_sc as plsc

_VOCAB = 100000
_HASH_SIZE = 4194304
_B = 4096
_L = 50
_NPAIR = _B * (_L - 1)
_NW = 32
_ROWS_W = _B // _NW
_NQ = 4
_ROWS_Q = _ROWS_W // _NQ
_PER_Q = _ROWS_Q * (_L - 1)
_CHUNKS = (0, 16, 32, 33)
_STAGE = 6256


def _lm_body(x_hbm, uni_hbm, bo_hbm, bg_hbm, fnd_hbm, out_hbm,
             x_v, out_v, bnc_u, bnc_b, uni_sh, bo_sh, *rest):
    bufs = rest[:7 * _NQ]
    sems = rest[7 * _NQ:]
    s_x, s_su, s_sb = sems[0], sems[1], sems[2]
    qsems = sems[3:]

    sid = lax.axis_index("s")
    wid = sid * 2 + lax.axis_index("c")
    row0 = wid * _ROWS_W

    sbase = jnp.minimum(sid * _STAGE, _VOCAB - _STAGE)
    cp_su = pltpu.async_copy(uni_hbm.at[pl.ds(sbase, _STAGE)], bnc_u, s_su)
    cp_sb = pltpu.async_copy(bo_hbm.at[pl.ds(sbase, _STAGE)], bnc_b, s_sb)
    cp_x = pltpu.async_copy(x_hbm.at[pl.ds(row0 * _L, _ROWS_W * _L)], x_v, s_x)

    with jax.named_scope("wait_x"):
        cp_x.wait()

    def unpack_quarter(r0, prev_r, cur_r, h_r):
        def body(r, _):
            for o in _CHUNKS:
                pv = x_v[pl.ds((r0 + r) * _L + o, 16)]
                cv = x_v[pl.ds((r0 + r) * _L + o + 1, 16)]
                sl = pl.ds(r * (_L - 1) + o, 16)
                prev_r[sl] = pv
                cur_r[sl] = cv
                h_r[sl] = (pv * 1000003 + cv) & (_HASH_SIZE - 1)
            return 0
        lax.fori_loop(0, _ROWS_Q, body, 0)

    def blend_quarter(r0, uni_r, bo_r, bg_r, fnd_r):
        def body(r, _):
            for o in _CHUNKS:
                sl = pl.ds(r * (_L - 1) + o, 16)
                f = fnd_r[sl]
                out_v[r0 + r, pl.ds(o, 16)] = (
                    f * bg_r[sl] + (1.0 - f) * (bo_r[sl] + uni_r[sl]))
            return 0
        lax.fori_loop(0, _ROWS_Q, body, 0)

    copies = []
    for q in range(_NQ):
        prev_r, cur_r, h_r, uni_r, bo_r, bg_r, fnd_r = bufs[7 * q:7 * q + 7]
        s_u, s_o, s_g, s_f = qsems[4 * q:4 * q + 4]
        with jax.named_scope("unpack"):
            unpack_quarter(q * _ROWS_Q, prev_r, cur_r, h_r)
        cp_g = pltpu.async_copy(bg_hbm.at[h_r], bg_r, s_g)
        cp_f = pltpu.async_copy(fnd_hbm.at[h_r], fnd_r, s_f)
        if q == 0:
            with jax.named_scope("publish"):
                cp_su.wait()
                pltpu.sync_copy(bnc_u, uni_sh.at[pl.ds(sbase, _STAGE)])
                cp_sb.wait()
                pltpu.sync_copy(bnc_b, bo_sh.at[pl.ds(sbase, _STAGE)])
            plsc.subcore_barrier()
        cp_u = pltpu.async_copy(uni_sh.at[cur_r], uni_r, s_u)
        cp_o = pltpu.async_copy(bo_sh.at[prev_r], bo_r, s_o)
        copies.append((cp_u, cp_o, cp_g, cp_f))

    for q in range(_NQ):
        _, _, _, uni_r, bo_r, bg_r, fnd_r = bufs[7 * q:7 * q + 7]
        cp_u, cp_o, cp_g, cp_f = copies[q]
        with jax.named_scope("wait_q"):
            cp_u.wait()
            cp_o.wait()
            cp_g.wait()
            cp_f.wait()
        with jax.named_scope("blend"):
            blend_quarter(q * _ROWS_Q, uni_r, bo_r, bg_r, fnd_r)

    with jax.named_scope("out"):
        pltpu.sync_copy(out_v, out_hbm.at[pl.ds(row0, _ROWS_W)])


@jax.jit
def _lm(x, uni, bo, bg, fnd):
    quarter_bufs = [
        pltpu.VMEM((_PER_Q,), jnp.int32),
        pltpu.VMEM((_PER_Q,), jnp.int32),
        pltpu.VMEM((_PER_Q,), jnp.int32),
        pltpu.VMEM((_PER_Q,), jnp.float32),
        pltpu.VMEM((_PER_Q,), jnp.float32),
        pltpu.VMEM((_PER_Q,), jnp.float32),
        pltpu.VMEM((_PER_Q,), jnp.float32),
    ]
    run = pl.kernel(
        _lm_body,
        out_type=jax.ShapeDtypeStruct((_B, _L - 1), jnp.float32),
        mesh=plsc.VectorSubcoreMesh(core_axis_name="c", subcore_axis_name="s"),
        scratch_types=(
            [pltpu.VMEM((_ROWS_W * _L,), jnp.int32),
             pltpu.VMEM((_ROWS_W, _L - 1), jnp.float32),
             pltpu.VMEM((_STAGE,), jnp.float32),
             pltpu.VMEM((_STAGE,), jnp.float32),
             pltpu.VMEM_SHARED((_VOCAB,), jnp.float32),
             pltpu.VMEM_SHARED((_VOCAB,), jnp.float32)]
            + quarter_bufs * _NQ
            + [pltpu.SemaphoreType.DMA] * (3 + 4 * _NQ)
        ),
    )
    return run(x, uni, bo, bg, fnd)


def kernel(x, unigram_logp, unigram_backoff, bigram_logp, bigram_found):
    return _lm(x.astype(jnp.int32).reshape(-1), unigram_logp,
               unigram_backoff, bigram_logp, bigram_found)


# speedup vs baseline: 1.0261x; 1.0261x over previous
import jax
import jax.numpy as jnp
from jax import lax
from jax.experimental import pallas as pl
from jax.experimental.pallas import tpu as pltpu
from jax.experimental.pallas import tpu_sc as plsc

_VOCAB = 100000
_HASH_SIZE = 4194304
_B = 4096
_L = 50
_NPAIR = _B * (_L - 1)
_NW = 32
_ROWS_W = _B // _NW
_NQ = 4
_ROWS_Q = _ROWS_W // _NQ
_PER_Q = _ROWS_Q * (_L - 1)
_CHUNKS = (0, 16, 32, 33)
_STAGE = 6256


def _lm_body(x_hbm, uni_hbm, bo_hbm, bg_hbm, fnd_hbm, out_hbm,
             x_v, out_v, bnc_u, bnc_b, uni_sh, bo_sh, *rest):
    bufs = rest[:7 * _NQ]
    sems = rest[7 * _NQ:]
    s_x, s_su, s_sb = sems[0], sems[1], sems[2]
    qsems = sems[3:]

    sid = lax.axis_index("s")
    wid = sid * 2 + lax.axis_index("c")
    row0 = wid * _ROWS_W

    sbase = jnp.minimum(sid * _STAGE, _VOCAB - _STAGE)
    cp_su = pltpu.async_copy(uni_hbm.at[pl.ds(sbase, _STAGE)], bnc_u, s_su)
    cp_sb = pltpu.async_copy(bo_hbm.at[pl.ds(sbase, _STAGE)], bnc_b, s_sb)
    cp_x = pltpu.async_copy(x_hbm.at[pl.ds(row0, _ROWS_W)], x_v, s_x)

    with jax.named_scope("wait_x"):
        cp_x.wait()

    def unpack_quarter(r0, prev_r, cur_r, h_r):
        def body(r, _):
            for o in _CHUNKS:
                pv = x_v[r0 + r, pl.ds(o, 16)]
                cv = x_v[r0 + r, pl.ds(o + 1, 16)]
                sl = pl.ds(r * (_L - 1) + o, 16)
                prev_r[sl] = pv
                cur_r[sl] = cv
                h_r[sl] = (pv * 1000003 + cv) & (_HASH_SIZE - 1)
            return 0
        lax.fori_loop(0, _ROWS_Q, body, 0)

    def blend_quarter(r0, uni_r, bo_r, bg_r, fnd_r):
        def body(r, _):
            for o in _CHUNKS:
                sl = pl.ds(r * (_L - 1) + o, 16)
                f = fnd_r[sl]
                out_v[r0 + r, pl.ds(o, 16)] = (
                    f * bg_r[sl] + (1.0 - f) * (bo_r[sl] + uni_r[sl]))
            return 0
        lax.fori_loop(0, _ROWS_Q, body, 0)

    copies = []
    for q in range(_NQ):
        prev_r, cur_r, h_r, uni_r, bo_r, bg_r, fnd_r = bufs[7 * q:7 * q + 7]
        s_u, s_o, s_g, s_f = qsems[4 * q:4 * q + 4]
        with jax.named_scope("unpack"):
            unpack_quarter(q * _ROWS_Q, prev_r, cur_r, h_r)
        cp_g = pltpu.async_copy(bg_hbm.at[h_r], bg_r, s_g)
        cp_f = pltpu.async_copy(fnd_hbm.at[h_r], fnd_r, s_f)
        if q == 0:
            with jax.named_scope("publish"):
                cp_su.wait()
                pltpu.sync_copy(bnc_u, uni_sh.at[pl.ds(sbase, _STAGE)])
                cp_sb.wait()
                pltpu.sync_copy(bnc_b, bo_sh.at[pl.ds(sbase, _STAGE)])
            plsc.subcore_barrier()
        cp_u = pltpu.async_copy(uni_sh.at[cur_r], uni_r, s_u)
        cp_o = pltpu.async_copy(bo_sh.at[prev_r], bo_r, s_o)
        copies.append((cp_u, cp_o, cp_g, cp_f))

    for q in range(_NQ):
        _, _, _, uni_r, bo_r, bg_r, fnd_r = bufs[7 * q:7 * q + 7]
        cp_u, cp_o, cp_g, cp_f = copies[q]
        with jax.named_scope("wait_q"):
            cp_u.wait()
            cp_o.wait()
            cp_g.wait()
            cp_f.wait()
        with jax.named_scope("blend"):
            blend_quarter(q * _ROWS_Q, uni_r, bo_r, bg_r, fnd_r)

    with jax.named_scope("out"):
        pltpu.sync_copy(out_v, out_hbm.at[pl.ds(row0, _ROWS_W)])


@jax.jit
def _lm(x, uni, bo, bg, fnd):
    quarter_bufs = [
        pltpu.VMEM((_PER_Q,), jnp.int32),
        pltpu.VMEM((_PER_Q,), jnp.int32),
        pltpu.VMEM((_PER_Q,), jnp.int32),
        pltpu.VMEM((_PER_Q,), jnp.float32),
        pltpu.VMEM((_PER_Q,), jnp.float32),
        pltpu.VMEM((_PER_Q,), jnp.float32),
        pltpu.VMEM((_PER_Q,), jnp.float32),
    ]
    run = pl.kernel(
        _lm_body,
        out_type=jax.ShapeDtypeStruct((_B, _L - 1), jnp.float32),
        mesh=plsc.VectorSubcoreMesh(core_axis_name="c", subcore_axis_name="s"),
        scratch_types=(
            [pltpu.VMEM((_ROWS_W, _L), jnp.int32),
             pltpu.VMEM((_ROWS_W, _L - 1), jnp.float32),
             pltpu.VMEM((_STAGE,), jnp.float32),
             pltpu.VMEM((_STAGE,), jnp.float32),
             pltpu.VMEM_SHARED((_VOCAB,), jnp.float32),
             pltpu.VMEM_SHARED((_VOCAB,), jnp.float32)]
            + quarter_bufs * _NQ
            + [pltpu.SemaphoreType.DMA] * (3 + 4 * _NQ)
        ),
    )
    return run(x, uni, bo, bg, fnd)


def kernel(x, unigram_logp, unigram_backoff, bigram_logp, bigram_found):
    return _lm(x.astype(jnp.int32), unigram_logp, unigram_backoff,
               bigram_logp, bigram_found)
